# Initial kernel scaffold; baseline (speedup 1.0000x reference)
import jax
import jax.numpy as jnp
from jax.experimental import pallas as pl


def kernel(x, edge_index, params):
    raise NotImplementedError("write your pallas kernel here")


# speedup vs baseline: 9.7438x; 9.7438x over previous
import functools

import jax
import jax.numpy as jnp
from jax import lax
from jax.experimental import pallas as pl
from jax.experimental.pallas import tpu as pltpu
from jax.experimental.pallas import tpu_sc as plsc

_N = 10000
_NP = 10240
_E = 320000
_EPAD = 327680
_ER = _EPAD // 128
_NC = 2
_NS = 16
_NPT = _NP // _NS
_RPT = _ER // (_NC * _NS)
_CHR = 16
_NCH = _RPT // _CHR
_H = 32
_BR = 1024
_GRID = _NP // _BR


_sc_mesh = plsc.VectorSubcoreMesh(
    core_axis_name="c", subcore_axis_name="s", num_cores=_NC, num_subcores=_NS)
_sc_params = pltpu.CompilerParams(use_tc_tiling_on_sc=False)


@functools.partial(
    pl.kernel,
    out_type=jax.ShapeDtypeStruct((_NC * _NP, _H), jnp.float32),
    mesh=_sc_mesh,
    compiler_params=_sc_params,
    scratch_types=[
        pltpu.VMEM((_CHR, 128), jnp.int32),
        pltpu.VMEM((_CHR, 128), jnp.int32),
        pltpu.VMEM((_CHR * 128, _H), jnp.float32),
        pltpu.VMEM((_NPT, _H), jnp.float32),
        pltpu.VMEM_SHARED((_NP, _H), jnp.float32),
        pltpu.SemaphoreType.DMA,
    ],
)
def _sc_segsum(feat_hbm, row_hbm, col_hbm, out_hbm, ridx, cidx, rows, wb, acc, sem):
    cid = lax.axis_index("c")
    sid = lax.axis_index("s")
    wid = cid * _NS + sid
    zv = jnp.zeros((16,), jnp.float32)

    def zero_row(i, carry):
        wb[i, pl.ds(0, 16)] = zv
        wb[i, pl.ds(16, 16)] = zv
        return carry

    lax.fori_loop(0, _NPT, zero_row, 0)
    pltpu.sync_copy(wb, acc.at[pl.ds(sid * _NPT, _NPT)])
    plsc.subcore_barrier()

    def chunk(k, carry):
        roff = wid * _RPT + k * _CHR
        pltpu.sync_copy(row_hbm.at[pl.ds(roff, _CHR)], ridx)
        pltpu.sync_copy(col_hbm.at[pl.ds(roff, _CHR)], cidx)
        cps = [
            pltpu.async_copy(feat_hbm.at[ridx.at[j]], rows.at[pl.ds(j * 128, 128)], sem)
            for j in range(_CHR)
        ]
        for c in cps:
            c.wait()
        for j in range(_CHR):
            pltpu.sync_copy(rows.at[pl.ds(j * 128, 128)], acc.at[cidx.at[j]], add=True)
        return carry

    lax.fori_loop(0, _NCH, chunk, 0)
    plsc.subcore_barrier()
    pltpu.sync_copy(
        acc.at[pl.ds(sid * _NPT, _NPT)],
        out_hbm.at[pl.ds(cid * _NP + sid * _NPT, _NPT)],
    )


@functools.partial(
    pl.kernel,
    out_type=jax.ShapeDtypeStruct((_NC * 2 * _NP, 16), jnp.float32),
    mesh=_sc_mesh,
    compiler_params=_sc_params,
    scratch_types=[
        pltpu.VMEM((_CHR, 128), jnp.int32),
        pltpu.VMEM((_CHR, 128), jnp.int32),
        pltpu.VMEM((128, 16), jnp.float32),
        pltpu.VMEM((_NPT, 16), jnp.float32),
        pltpu.VMEM_SHARED((_NP, 16), jnp.float32),
        pltpu.VMEM_SHARED((_NP, 16), jnp.float32),
        pltpu.SemaphoreType.DMA,
    ],
)
def _sc_degrees(row_hbm, col_hbm, out_hbm, ridx, cidx, ones_v, wb, accc, accd, sem):
    cid = lax.axis_index("c")
    sid = lax.axis_index("s")
    wid = cid * _NS + sid
    zv = jnp.zeros((16,), jnp.float32)
    ov = jnp.ones((16,), jnp.float32)

    def zero_row(i, carry):
        wb[i, pl.ds(0, 16)] = zv
        return carry

    lax.fori_loop(0, _NPT, zero_row, 0)

    def ones_row(i, carry):
        ones_v[i, pl.ds(0, 16)] = ov
        return carry

    lax.fori_loop(0, 128, ones_row, 0)
    pltpu.sync_copy(wb, accc.at[pl.ds(sid * _NPT, _NPT)])
    pltpu.sync_copy(wb, accd.at[pl.ds(sid * _NPT, _NPT)])
    plsc.subcore_barrier()

    def chunk(k, carry):
        roff = wid * _RPT + k * _CHR
        pltpu.sync_copy(row_hbm.at[pl.ds(roff, _CHR)], ridx)
        pltpu.sync_copy(col_hbm.at[pl.ds(roff, _CHR)], cidx)
        for j in range(_CHR):
            pltpu.sync_copy(ones_v, accc.at[cidx.at[j]], add=True)
            pltpu.sync_copy(ones_v, accd.at[ridx.at[j]], add=True)
        return carry

    lax.fori_loop(0, _NCH, chunk, 0)
    plsc.subcore_barrier()
    base = cid * 2 * _NP
    pltpu.sync_copy(
        accc.at[pl.ds(sid * _NPT, _NPT)], out_hbm.at[pl.ds(base + sid * _NPT, _NPT)]
    )
    pltpu.sync_copy(
        accd.at[pl.ds(sid * _NPT, _NPT)],
        out_hbm.at[pl.ds(base + _NP + sid * _NPT, _NPT)],
    )


def _rb(w):
    return pl.BlockSpec((_BR, w), lambda i: (i, 0))


def _full(r, c):
    return pl.BlockSpec((r, c), lambda i: (0, 0))


def _proj_body(x_ref, w_ref, b_ref, skip_ref, res_ref, p0_ref, xr0_ref):
    o = jnp.dot(x_ref[...], w_ref[...], preferred_element_type=jnp.float32) + b_ref[...]
    skip_ref[...] = o[:, 0:32]
    res_ref[...] = o[:, 32:64]
    p0_ref[...] = o[:, 64:96]
    xr0_ref[...] = o[:, 96:128]


_tc_proj = pl.pallas_call(
    _proj_body,
    grid=(_GRID,),
    in_specs=[_rb(128), _full(128, 128), _full(1, 128)],
    out_specs=[_rb(_H)] * 4,
    out_shape=[jax.ShapeDtypeStruct((_NP, _H), jnp.float32)] * 4,
)


def _sage_body(y0_ref, y1_ref, c0_ref, c1_ref, xr_ref, hres_ref, w_ref, s_ref,
               t_ref, h_ref, p_ref, xr2_ref):
    rin = 1.0 / jnp.maximum(c0_ref[:, 0:1] + c1_ref[:, 0:1], 1.0)
    sage = (y0_ref[...] + y1_ref[...]) * rin + xr_ref[...]
    h = jnp.maximum(sage * s_ref[...] + t_ref[...], 0.0) + hres_ref[...]
    h_ref[...] = h
    pq = jnp.dot(h, w_ref[...], preferred_element_type=jnp.float32)
    p_ref[...] = pq[:, 0:_H]
    xr2_ref[...] = pq[:, _H:2 * _H]


_tc_sage = pl.pallas_call(
    _sage_body,
    grid=(_GRID,),
    in_specs=[_rb(_H), _rb(_H), _rb(16), _rb(16), _rb(_H), _rb(_H),
              _full(_H, 2 * _H), _full(1, _H), _full(1, _H)],
    out_specs=[_rb(_H)] * 3,
    out_shape=[jax.ShapeDtypeStruct((_NP, _H), jnp.float32)] * 3,
)


def _sage4_body(y0_ref, y1_ref, c0_ref, c1_ref, d0_ref, d1_ref, xr_ref,
                hres_ref, s_ref, t_ref, h_ref, hs_ref):
    rin = 1.0 / jnp.maximum(c0_ref[:, 0:1] + c1_ref[:, 0:1], 1.0)
    rdeg = 1.0 / jnp.maximum(d0_ref[:, 0:1] + d1_ref[:, 0:1], 1.0)
    sage = (y0_ref[...] + y1_ref[...]) * rin + xr_ref[...]
    h = jnp.maximum(sage * s_ref[...] + t_ref[...], 0.0) + hres_ref[...]
    h_ref[...] = h
    hs_ref[...] = h * rdeg


_tc_sage4 = pl.pallas_call(
    _sage4_body,
    grid=(_GRID,),
    in_specs=[_rb(_H), _rb(_H), _rb(16), _rb(16), _rb(16), _rb(16), _rb(_H),
              _rb(_H), _full(1, _H), _full(1, _H)],
    out_specs=[_rb(_H)] * 2,
    out_shape=[jax.ShapeDtypeStruct((_NP, _H), jnp.float32)] * 2,
)


def _path_body(y0_ref, y1_ref, d0_ref, d1_ref, o_ref):
    rdeg = 1.0 / jnp.maximum(d0_ref[:, 0:1] + d1_ref[:, 0:1], 1.0)
    o_ref[...] = (y0_ref[...] + y1_ref[...]) * rdeg


_tc_path = pl.pallas_call(
    _path_body,
    grid=(_GRID,),
    in_specs=[_rb(_H), _rb(_H), _rb(16), _rb(16)],
    out_specs=_rb(_H),
    out_shape=jax.ShapeDtypeStruct((_NP, _H), jnp.float32),
)


def _vgaein_body(h1_ref, h2_ref, h3_ref, h4_ref, sk_ref, y0_ref, y1_ref,
                 w_ref, pv_ref, xrv_ref):
    hagg = y0_ref[...] + y1_ref[...]
    cat = jnp.concatenate(
        [h1_ref[...], h2_ref[...], h3_ref[...], h4_ref[...], sk_ref[...], hagg],
        axis=1)
    pq = jnp.dot(cat, w_ref[...], preferred_element_type=jnp.float32)
    pv_ref[...] = pq[:, 0:_H]
    xrv_ref[...] = pq[:, _H:2 * _H]


_tc_vgaein = pl.pallas_call(
    _vgaein_body,
    grid=(_GRID,),
    in_specs=[_rb(_H)] * 7 + [_full(192, 2 * _H)],
    out_specs=[_rb(_H)] * 2,
    out_shape=[jax.ShapeDtypeStruct((_NP, _H), jnp.float32)] * 2,
)


def _tail_body(y0_ref, y1_ref, c0_ref, c1_ref, xrv_ref,
               h1_ref, h2_ref, h3_ref, h4_ref, sk_ref,
               sv_ref, tv_ref, wml_ref, bml_ref, wr1_ref, br1_ref, wr2_ref,
               br2_ref, wg1a_ref, wg1b_ref, wg1c_ref, bg1_ref, wg2_ref,
               bg2_ref, wg3_ref, bg3_ref,
               preds_ref, rank_ref, mu_ref, lv_ref):
    rin = 1.0 / jnp.maximum(c0_ref[:, 0:1] + c1_ref[:, 0:1], 1.0)
    sage = (y0_ref[...] + y1_ref[...]) * rin + xrv_ref[...]
    vf = jnp.maximum(sage * sv_ref[...] + tv_ref[...], 0.0)
    ml = jnp.dot(vf, wml_ref[...], preferred_element_type=jnp.float32) + bml_ref[...]
    mu = ml[:, 0:_H]
    mu_ref[...] = mu
    lv_ref[...] = ml[:, _H:2 * _H]
    r = jnp.maximum(
        jnp.dot(mu, wr1_ref[...], preferred_element_type=jnp.float32) + br1_ref[...],
        0.0)
    rank = jnp.dot(r, wr2_ref[...], preferred_element_type=jnp.float32) + br2_ref[...]
    rank_ref[...] = rank
    s1 = jnp.concatenate(
        [h1_ref[...], h2_ref[...], h3_ref[...], h4_ref[...], sk_ref[...]], axis=1)
    g = jnp.dot(s1, wg1a_ref[...], preferred_element_type=jnp.float32)
    g = g + jnp.dot(mu, wg1b_ref[...], preferred_element_type=jnp.float32)
    g = jnp.maximum(g + rank * wg1c_ref[...] + bg1_ref[...], 0.0)
    g = jnp.maximum(
        jnp.dot(g, wg2_ref[...], preferred_element_type=jnp.float32) + bg2_ref[...],
        0.0)
    preds_ref[...] = (
        jnp.dot(g, wg3_ref[...], preferred_element_type=jnp.float32) + bg3_ref[...])


_tc_tail = pl.pallas_call(
    _tail_body,
    grid=(_GRID,),
    in_specs=[_rb(_H), _rb(_H), _rb(16), _rb(16), _rb(_H)]
    + [_rb(_H)] * 5
    + [_full(1, _H), _full(1, _H), _full(_H, 2 * _H), _full(1, 2 * _H),
       _full(_H, _H), _full(1, _H), _full(_H, 1), _full(1, 1),
       _full(160, _H), _full(_H, _H), _full(1, _H), _full(1, _H),
       _full(_H, 16), _full(1, 16), _full(16, 1), _full(1, 1)],
    out_specs=[_rb(1), _rb(1), _rb(_H), _rb(_H)],
    out_shape=[jax.ShapeDtypeStruct((_NP, 1), jnp.float32),
               jax.ShapeDtypeStruct((_NP, 1), jnp.float32),
               jax.ShapeDtypeStruct((_NP, _H), jnp.float32),
               jax.ShapeDtypeStruct((_NP, _H), jnp.float32)],
)


def _bn_affine(bp, eps=1e-5):
    s = bp["g"] * lax.rsqrt(bp["v"] + eps)
    t = bp["beta"] - bp["m"] * s
    return s, t


def kernel(x, edge_index, params):
    p = params
    x_p = jnp.pad(x, ((0, _NP - _N), (0, 0)))
    pad_idx = jnp.full((_EPAD - _E,), _N, jnp.int32)
    row2 = jnp.concatenate([edge_index[0], pad_idx]).reshape(_ER, 128)
    col2 = jnp.concatenate([edge_index[1], pad_idx]).reshape(_ER, 128)

    wcat0 = jnp.concatenate(
        [p["input_proj"]["W"].T, p["res0"]["W"].T,
         p["convs"][0]["Wl"].T, p["convs"][0]["Wr"].T], axis=1)
    bcat0 = jnp.concatenate(
        [p["input_proj"]["b"], p["res0"]["b"],
         jnp.zeros((2 * _H,), jnp.float32)])[None, :]
    skip, res, pk, xrk = _tc_proj(x_p, wcat0, bcat0)

    degs = _sc_degrees(row2, col2)
    cnt0, deg0 = degs[0:_NP], degs[_NP:2 * _NP]
    cnt1, deg1 = degs[2 * _NP:3 * _NP], degs[3 * _NP:4 * _NP]

    hs_list = []
    hres = res
    for i in range(3):
        ypart = _sc_segsum(pk, row2, col2)
        s_i, t_i = _bn_affine(p["bns"][i])
        t2 = (t_i + p["convs"][i]["b"] * s_i)[None, :]
        wnext = jnp.concatenate(
            [p["convs"][i + 1]["Wl"].T, p["convs"][i + 1]["Wr"].T], axis=1)
        h, pk, xrk = _tc_sage(ypart[0:_NP], ypart[_NP:2 * _NP], cnt0, cnt1,
                              xrk, hres, wnext, s_i[None, :], t2)
        hs_list.append(h)
        hres = h

    ypart = _sc_segsum(pk, row2, col2)
    s_i, t_i = _bn_affine(p["bns"][3])
    t2 = (t_i + p["convs"][3]["b"] * s_i)[None, :]
    h4, hsc = _tc_sage4(ypart[0:_NP], ypart[_NP:2 * _NP], cnt0, cnt1, deg0,
                        deg1, xrk, hres, s_i[None, :], t2)
    hs_list.append(h4)

    for _ in range(3):
        ypart = _sc_segsum(hsc, row2, col2)
        hsc = _tc_path(ypart[0:_NP], ypart[_NP:2 * _NP], deg0, deg1)
    ypart = _sc_segsum(hsc, row2, col2)

    wv = jnp.concatenate([p["vgae_conv"]["Wl"].T, p["vgae_conv"]["Wr"].T], axis=1)
    pv, xrv = _tc_vgaein(hs_list[0], hs_list[1], hs_list[2], hs_list[3], skip,
                         ypart[0:_NP], ypart[_NP:2 * _NP], wv)
    ypart = _sc_segsum(pv, row2, col2)

    sv, tv = _bn_affine(p["vgae_bn"])
    tv2 = (tv + p["vgae_conv"]["b"] * sv)[None, :]
    wml = jnp.concatenate([p["mu"]["W"].T, p["logvar"]["W"].T], axis=1)
    bml = jnp.concatenate([p["mu"]["b"], p["logvar"]["b"]])[None, :]
    wg1 = p["reg"][0]["W"].T
    preds, rank, mu, lv = _tc_tail(
        ypart[0:_NP], ypart[_NP:2 * _NP], cnt0, cnt1, xrv,
        hs_list[0], hs_list[1], hs_list[2], hs_list[3], skip,
        sv[None, :], tv2, wml, bml,
        p["rank"][0]["W"].T, p["rank"][0]["b"][None, :],
        p["rank"][1]["W"].T, p["rank"][1]["b"][None, :],
        wg1[0:160], wg1[160:192], wg1[192:193], p["reg"][0]["b"][None, :],
        p["reg"][1]["W"].T, p["reg"][1]["b"][None, :],
        p["reg"][2]["W"].T, p["reg"][2]["b"][None, :])

    return (preds[:_N], rank[:_N, 0], mu[:_N], lv[:_N])
